# Initial kernel scaffold; baseline (speedup 1.0000x reference)
import jax
import jax.numpy as jnp
from jax.experimental import pallas as pl


def kernel(user_features, embedding_table):
    raise NotImplementedError("write your pallas kernel here")


# speedup vs baseline: 3.2467x; 3.2467x over previous
import functools

import jax
import jax.numpy as jnp
from jax import lax
from jax.experimental import pallas as pl
from jax.experimental.pallas import tpu as pltpu
from jax.experimental.pallas import tpu_sc as plsc

_VOCAB = 1000
_D = 16
_B = 16384
_F = 26

_NC = 2
_NS = 16
_NW = _NC * _NS

_ROWS = _B * _F
_ROWS_W = _ROWS // _NW
_IDX_BLK = 128
_NBLK_W = _ROWS_W // _IDX_BLK

_NB = 64
_CHUNK = _NB * _F
_BLK_PER_CHUNK = _CHUNK // _IDX_BLK
_NCHUNK = _ROWS_W // _CHUNK


def _rsqrt(x):
    i = lax.bitcast_convert_type(x, jnp.int32)
    i = jnp.int32(0x5F3759DF) - (i >> 1)
    y = lax.bitcast_convert_type(i, jnp.float32)
    for _ in range(3):
        y = y * (jnp.float32(1.5) - jnp.float32(0.5) * x * y * y)
    return y


def _body(table_hbm, idx_hbm, out_hbm, idx_v, buf, gsem):
    wid = lax.axis_index("s") * _NC + lax.axis_index("c")

    pltpu.sync_copy(idx_hbm.at[pl.ds(wid * _NBLK_W, _NBLK_W)], idx_v)

    def normalize_row(r, _):
        base = r * _F
        vs = [buf[base + f] for f in range(_F)]
        acc = vs[0] * vs[0]
        for f in range(1, _F):
            acc = acc + vs[f] * vs[f]
        norm = acc * _rsqrt(acc)
        recip = jnp.float32(1.0) / jnp.maximum(norm, jnp.float32(1e-12))
        for f in range(_F):
            buf[base + f] = vs[f] * recip
        return _

    for c in range(_NCHUNK):
        copies = []
        for j in range(_BLK_PER_CHUNK):
            copies.append(pltpu.async_copy(
                table_hbm.at[idx_v.at[c * _BLK_PER_CHUNK + j]],
                buf.at[pl.ds(j * _IDX_BLK, _IDX_BLK)],
                gsem))
        for cp in copies:
            cp.wait()
        lax.fori_loop(0, _NB, normalize_row, None)
        pltpu.sync_copy(
            buf, out_hbm.at[pl.ds(wid * _ROWS_W + c * _CHUNK, _CHUNK)])


def kernel(user_features, embedding_table):
    idx2d = user_features.reshape(_ROWS // _IDX_BLK, _IDX_BLK)
    mesh = plsc.VectorSubcoreMesh(
        core_axis_name="c", subcore_axis_name="s",
        num_cores=_NC, num_subcores=_NS)
    run = functools.partial(
        pl.kernel,
        out_type=jax.ShapeDtypeStruct((_ROWS, _D), jnp.float32),
        mesh=mesh,
        scratch_types=[
            pltpu.VMEM((_NBLK_W, _IDX_BLK), jnp.int32),
            pltpu.VMEM((_CHUNK, _D), jnp.float32),
            pltpu.SemaphoreType.DMA,
        ],
        compiler_params=pltpu.CompilerParams(use_tc_tiling_on_sc=False),
    )(_body)
    out = run(embedding_table, idx2d)
    return out.reshape(_B, _F, _D)


# speedup vs baseline: 5.0624x; 1.5593x over previous
import functools

import jax
import jax.numpy as jnp
from jax import lax
from jax.experimental import pallas as pl
from jax.experimental.pallas import tpu as pltpu
from jax.experimental.pallas import tpu_sc as plsc

_VOCAB = 1000
_D = 16
_B = 16384
_F = 26

_NC = 2
_NS = 16
_NW = _NC * _NS

_ROWS_W = _B // _NW
_NB = 16
_NCHUNK = _ROWS_W // _NB


def _rsqrt(x):
    i = lax.bitcast_convert_type(x, jnp.int32)
    i = jnp.int32(0x5F3759DF) - (i >> 1)
    y = lax.bitcast_convert_type(i, jnp.float32)
    for _ in range(3):
        y = y * (jnp.float32(1.5) - jnp.float32(0.5) * x * y * y)
    return y


def _body(table_hbm, idx_hbm, out_hbm, idx_v, buf, gsem):
    wid = lax.axis_index("s") * _NC + lax.axis_index("c")

    pltpu.sync_copy(idx_hbm.at[pl.ds(wid * _ROWS_W, _ROWS_W)], idx_v)

    def normalize_row(r, _):
        vs = [buf[r, f] for f in range(_F)]
        acc = vs[0] * vs[0]
        for f in range(1, _F):
            acc = acc + vs[f] * vs[f]
        norm = acc * _rsqrt(acc)
        recip = jnp.float32(1.0) / jnp.maximum(norm, jnp.float32(1e-12))
        for f in range(_F):
            buf[r, f] = vs[f] * recip
        return _

    def chunk(c, _):
        row0 = c * _NB
        copies = []
        for r in range(_NB):
            copies.append(pltpu.async_copy(
                table_hbm.at[idx_v.at[row0 + r]], buf.at[r], gsem))
        for cp in copies:
            cp.wait()
        lax.fori_loop(0, _NB, normalize_row, None)
        pltpu.sync_copy(buf, out_hbm.at[pl.ds(wid * _ROWS_W + row0, _NB)])
        return _

    lax.fori_loop(0, _NCHUNK, chunk, None)


def kernel(user_features, embedding_table):
    mesh = plsc.VectorSubcoreMesh(
        core_axis_name="c", subcore_axis_name="s",
        num_cores=_NC, num_subcores=_NS)
    run = functools.partial(
        pl.kernel,
        out_type=jax.ShapeDtypeStruct((_B, _F, _D), jnp.float32),
        mesh=mesh,
        scratch_types=[
            pltpu.VMEM((_ROWS_W, _F), jnp.int32),
            pltpu.VMEM((_NB, _F, _D), jnp.float32),
            pltpu.SemaphoreType.DMA,
        ],
        compiler_params=pltpu.CompilerParams(use_tc_tiling_on_sc=False),
    )(_body)
    return run(embedding_table, user_features)
